# Initial kernel scaffold; baseline (speedup 1.0000x reference)
import jax
import jax.numpy as jnp
from jax.experimental import pallas as pl


def kernel(pred, y, edge_capacity, edge_flow, net_demand, edge_index):
    raise NotImplementedError("write your pallas kernel here")


# speedup vs baseline: 11.0538x; 11.0538x over previous
import functools

import jax
import jax.numpy as jnp
from jax import lax
from jax.experimental import pallas as pl
from jax.experimental.pallas import tpu as pltpu
from jax.experimental.pallas import tpu_sc as plsc

N_NODES = 100000
N_EDGES = 3200000

W_VCR = 1.0
W_FLOW = 0.005
W_CONS = 0.05

TGT_SCALE, TGT_SHIFT = 0.8, 0.5
CAP_SCALE, CAP_SHIFT = 1500.0, 2000.0
FLOW_SCALE, FLOW_SHIFT = 1200.0, 0.0
DEM_SCALE, DEM_SHIFT = 500.0, 0.0

NW = 32
LANE = 128
ROWS = N_EDGES // LANE
CR = 8
CE = CR * LANE
NCHUNKS = ROWS // CR
CHUNK_LO = NCHUNKS // NW
CHUNK_EXTRA = NCHUNKS - CHUNK_LO * NW
NODES_PAD = 100096
SLICE = NODES_PAD // 16


def _k1_body(pred_h, y_h, cap_h, flow_h, src_h, dst_h,
             acc_o, part_o,
             pv, yv, cv, fv, pfv, npfv, sv, dv, stage, zbuf, acc_sh):
    cid = lax.axis_index("c")
    sid = lax.axis_index("s")
    w = cid * 16 + sid

    def zstep(i, carry):
        zbuf[pl.ds(i * 16, 16)] = jnp.zeros((16,), jnp.float32)
        return carry

    lax.fori_loop(0, SLICE // 16, zstep, 0)
    pltpu.sync_copy(zbuf, acc_sh.at[pl.ds(sid * SLICE, SLICE)])
    plsc.subcore_barrier()

    wmin = jnp.minimum(w, CHUNK_EXTRA)
    base_chunk = CHUNK_LO * w + wmin
    n_chunks = CHUNK_LO + jnp.where(w < CHUNK_EXTRA, 1, 0)

    def chunk(ci, carry):
        avcr, aflow = carry
        r0 = (base_chunk + ci) * CR
        e0 = r0 * LANE
        pltpu.sync_copy(pred_h.at[pl.ds(e0, CE)], pv)
        pltpu.sync_copy(y_h.at[pl.ds(e0, CE)], yv)
        pltpu.sync_copy(cap_h.at[pl.ds(e0, CE)], cv)
        pltpu.sync_copy(flow_h.at[pl.ds(e0, CE)], fv)
        pltpu.sync_copy(src_h.at[pl.ds(r0, CR)], sv)
        pltpu.sync_copy(dst_h.at[pl.ds(r0, CR)], dv)

        def estep(j, c2):
            av, af = c2
            p = pv[pl.ds(j * 16, 16)]
            yy = yv[pl.ds(j * 16, 16)]
            cc = cv[pl.ds(j * 16, 16)]
            ff = fv[pl.ds(j * 16, 16)]
            pf = (p * TGT_SCALE + TGT_SHIFT) * (cc * CAP_SCALE + CAP_SHIFT)
            d1 = p - yy
            d2 = pf - (ff * FLOW_SCALE + FLOW_SHIFT)
            pfv[pl.ds(j * 16, 16)] = pf
            npfv[pl.ds(j * 16, 16)] = -pf
            return (av + d1 * d1, af + d2 * d2)

        avcr, aflow = lax.fori_loop(0, CE // 16, estep, (avcr, aflow))

        for k in range(CR):
            pltpu.sync_copy(pfv.at[pl.ds(k * LANE, LANE)],
                            acc_sh.at[dv.at[k]], add=True)
            pltpu.sync_copy(npfv.at[pl.ds(k * LANE, LANE)],
                            acc_sh.at[sv.at[k]], add=True)
        return (avcr, aflow)

    zero16 = jnp.zeros((16,), jnp.float32)
    avcr, aflow = lax.fori_loop(0, n_chunks, chunk, (zero16, zero16))

    stage[...] = avcr
    pltpu.sync_copy(stage, part_o.at[pl.ds(w * 16, 16)])
    stage[...] = aflow
    pltpu.sync_copy(stage, part_o.at[pl.ds((NW + w) * 16, 16)])

    plsc.subcore_barrier()
    pltpu.sync_copy(acc_sh.at[pl.ds(sid * SLICE, SLICE)], zbuf)
    pltpu.sync_copy(zbuf, acc_o.at[pl.ds(cid * NODES_PAD + sid * SLICE, SLICE)])


_k1 = functools.partial(
    pl.kernel,
    mesh=plsc.VectorSubcoreMesh(core_axis_name="c", subcore_axis_name="s",
                                num_cores=2),
    out_type=[
        jax.ShapeDtypeStruct((2 * NODES_PAD,), jnp.float32),
        jax.ShapeDtypeStruct((2 * NW * 16,), jnp.float32),
    ],
    scratch_types=[
        pltpu.VMEM((CE,), jnp.float32),
        pltpu.VMEM((CE,), jnp.float32),
        pltpu.VMEM((CE,), jnp.float32),
        pltpu.VMEM((CE,), jnp.float32),
        pltpu.VMEM((CE,), jnp.float32),
        pltpu.VMEM((CE,), jnp.float32),
        pltpu.VMEM((CR, LANE), jnp.int32),
        pltpu.VMEM((CR, LANE), jnp.int32),
        pltpu.VMEM((16,), jnp.float32),
        pltpu.VMEM((SLICE,), jnp.float32),
        pltpu.VMEM_SHARED((NODES_PAD,), jnp.float32),
    ],
)(_k1_body)


def _k2_body(acc_ref, nd_ref, part_ref, out_ref):
    delta = acc_ref[0] + acc_ref[1]
    rnd = nd_ref[...] * DEM_SCALE + DEM_SHIFT
    cons = jnp.sum(jnp.abs(delta - rnd))
    vcr = jnp.sum(part_ref[0:NW, :])
    flw = jnp.sum(part_ref[NW:2 * NW, :])
    total = (W_VCR * vcr / N_EDGES
             + W_FLOW * flw / N_EDGES
             + W_CONS * cons / N_NODES)
    out_ref[...] = jnp.reshape(total, (1, 1))


_k2 = pl.pallas_call(
    _k2_body,
    out_shape=jax.ShapeDtypeStruct((1, 1), jnp.float32),
)


def kernel(pred, y, edge_capacity, edge_flow, net_demand, edge_index):
    ei = edge_index.astype(jnp.int32)
    src2 = ei[0].reshape(ROWS, LANE)
    dst2 = ei[1].reshape(ROWS, LANE)
    acc, part = _k1(pred, y, edge_capacity, edge_flow, src2, dst2)
    nd2 = jnp.pad(net_demand, (0, NODES_PAD - N_NODES)).reshape(NODES_PAD // LANE, LANE)
    out = _k2(acc.reshape(2, NODES_PAD // LANE, LANE), nd2,
              part.reshape(2 * NW, 16))
    return out[0, 0]


# speedup vs baseline: 42.5632x; 3.8505x over previous
import functools

import jax
import jax.numpy as jnp
from jax import lax
from jax.experimental import pallas as pl
from jax.experimental.pallas import tpu as pltpu
from jax.experimental.pallas import tpu_sc as plsc

N_NODES = 100000
N_EDGES = 3200000

W_VCR = 1.0
W_FLOW = 0.005
W_CONS = 0.05

TGT_SCALE, TGT_SHIFT = 0.8, 0.5
CAP_SCALE, CAP_SHIFT = 1500.0, 2000.0
FLOW_SCALE, FLOW_SHIFT = 1200.0, 0.0
DEM_SCALE, DEM_SHIFT = 500.0, 0.0

NW = 32
LANE = 128
ROWS = N_EDGES // LANE
CR = 8
CE = CR * LANE
NCHUNKS = ROWS // CR
CHUNK_LO = NCHUNKS // NW
CHUNK_EXTRA = NCHUNKS - CHUNK_LO * NW
NODES_PAD = 100096
SLICE = NODES_PAD // 16


def _k1_body(pred_h, y_h, cap_h, flow_h, src_h, dst_h,
             acc_o, part_o,
             pv0, yv0, cv0, fv0, pfv0, npfv0, sv0, dv0,
             pv1, yv1, cv1, fv1, pfv1, npfv1, sv1, dv1,
             stage, zbuf, acc_sh,
             isem0, isem1, ssem0, ssem1):
    cid = lax.axis_index("c")
    sid = lax.axis_index("s")
    w = cid * 16 + sid
    bufs = [(pv0, yv0, cv0, fv0, pfv0, npfv0, sv0, dv0, isem0, ssem0),
            (pv1, yv1, cv1, fv1, pfv1, npfv1, sv1, dv1, isem1, ssem1)]

    def zstep(i, carry):
        zbuf[pl.ds(i * 16, 16)] = jnp.zeros((16,), jnp.float32)
        return carry

    lax.fori_loop(0, SLICE // 16, zstep, 0)
    pltpu.sync_copy(zbuf, acc_sh.at[pl.ds(sid * SLICE, SLICE)])
    plsc.subcore_barrier()

    wmin = jnp.minimum(w, CHUNK_EXTRA)
    base_chunk = CHUNK_LO * w + wmin
    n_chunks = CHUNK_LO + jnp.where(w < CHUNK_EXTRA, 1, 0)

    def prefetch(g, s):
        pv, yv, cv, fv, _, _, sv, dv, isem, _ = bufs[s]
        e0 = g * CE
        r0 = g * CR
        pltpu.async_copy(pred_h.at[pl.ds(e0, CE)], pv, isem)
        pltpu.async_copy(y_h.at[pl.ds(e0, CE)], yv, isem)
        pltpu.async_copy(cap_h.at[pl.ds(e0, CE)], cv, isem)
        pltpu.async_copy(flow_h.at[pl.ds(e0, CE)], fv, isem)
        pltpu.async_copy(src_h.at[pl.ds(r0, CR)], sv, isem)
        pltpu.async_copy(dst_h.at[pl.ds(r0, CR)], dv, isem)

    def wait_in(s):
        pv, yv, cv, fv, _, _, sv, dv, isem, _ = bufs[s]
        for dst in (pv, yv, cv, fv):
            pltpu.make_async_copy(pred_h.at[pl.ds(0, CE)], dst, isem).wait()
        for dst in (sv, dv):
            pltpu.make_async_copy(src_h.at[pl.ds(0, CR)], dst, isem).wait()

    def compute(s, mvec, carry):
        pv, yv, cv, fv, pfv, npfv, _, _, _, _ = bufs[s]

        def estep(j, c2):
            av, af = c2
            p = pv[pl.ds(j * 16, 16)]
            yy = yv[pl.ds(j * 16, 16)]
            cc = cv[pl.ds(j * 16, 16)]
            ff = fv[pl.ds(j * 16, 16)]
            pf = (p * TGT_SCALE + TGT_SHIFT) * (cc * CAP_SCALE + CAP_SHIFT)
            d1 = p - yy
            d2 = pf - (ff * FLOW_SCALE + FLOW_SHIFT)
            if mvec is not None:
                pf = pf * mvec
                d1 = d1 * mvec
                d2 = d2 * mvec
            pfv[pl.ds(j * 16, 16)] = pf
            npfv[pl.ds(j * 16, 16)] = -pf
            return (av + d1 * d1, af + d2 * d2)

        return lax.fori_loop(0, CE // 16, estep, carry)

    def scatter(s):
        _, _, _, _, pfv, npfv, sv, dv, _, ssem = bufs[s]
        descs = []
        for k in range(CR):
            descs.append(pltpu.async_copy(pfv.at[pl.ds(k * LANE, LANE)],
                                          acc_sh.at[dv.at[k]], ssem, add=True))
            descs.append(pltpu.async_copy(npfv.at[pl.ds(k * LANE, LANE)],
                                          acc_sh.at[sv.at[k]], ssem, add=True))
        for d in descs:
            d.wait()

    prefetch(base_chunk, 0)

    def body(k, carry):
        g1 = jnp.minimum(base_chunk + 2 * k + 1, NCHUNKS - 1)
        prefetch(g1, 1)
        wait_in(0)
        carry = compute(0, None, carry)
        scatter(0)
        g2 = jnp.minimum(base_chunk + 2 * k + 2, NCHUNKS - 1)
        prefetch(g2, 0)
        wait_in(1)
        mvec = jnp.zeros((16,), jnp.float32) + jnp.where(
            2 * k + 1 < n_chunks, 1.0, 0.0).astype(jnp.float32)
        carry = compute(1, mvec, carry)
        scatter(1)
        return carry

    zero16 = jnp.zeros((16,), jnp.float32)
    avcr, aflow = lax.fori_loop(0, 49, body, (zero16, zero16))
    wait_in(0)

    stage[...] = avcr
    pltpu.sync_copy(stage, part_o.at[pl.ds(w * 16, 16)])
    stage[...] = aflow
    pltpu.sync_copy(stage, part_o.at[pl.ds((NW + w) * 16, 16)])

    plsc.subcore_barrier()
    pltpu.sync_copy(acc_sh.at[pl.ds(sid * SLICE, SLICE)], zbuf)
    pltpu.sync_copy(zbuf, acc_o.at[pl.ds(cid * NODES_PAD + sid * SLICE, SLICE)])


_k1 = functools.partial(
    pl.kernel,
    mesh=plsc.VectorSubcoreMesh(core_axis_name="c", subcore_axis_name="s",
                                num_cores=2),
    out_type=[
        jax.ShapeDtypeStruct((2 * NODES_PAD,), jnp.float32),
        jax.ShapeDtypeStruct((2 * NW * 16,), jnp.float32),
    ],
    scratch_types=(
        [pltpu.VMEM((CE,), jnp.float32)] * 6
        + [pltpu.VMEM((CR, LANE), jnp.int32)] * 2
        + [pltpu.VMEM((CE,), jnp.float32)] * 6
        + [pltpu.VMEM((CR, LANE), jnp.int32)] * 2
        + [
            pltpu.VMEM((16,), jnp.float32),
            pltpu.VMEM((SLICE,), jnp.float32),
            pltpu.VMEM_SHARED((NODES_PAD,), jnp.float32),
            pltpu.SemaphoreType.DMA,
            pltpu.SemaphoreType.DMA,
            pltpu.SemaphoreType.DMA,
            pltpu.SemaphoreType.DMA,
        ]
    ),
)(_k1_body)


def _k2_body(acc_ref, nd_ref, part_ref, out_ref):
    delta = acc_ref[0] + acc_ref[1]
    rnd = nd_ref[...] * DEM_SCALE + DEM_SHIFT
    cons = jnp.sum(jnp.abs(delta - rnd))
    vcr = jnp.sum(part_ref[0:NW, :])
    flw = jnp.sum(part_ref[NW:2 * NW, :])
    total = (W_VCR * vcr / N_EDGES
             + W_FLOW * flw / N_EDGES
             + W_CONS * cons / N_NODES)
    out_ref[...] = jnp.reshape(total, (1, 1))


_k2 = pl.pallas_call(
    _k2_body,
    out_shape=jax.ShapeDtypeStruct((1, 1), jnp.float32),
)


def kernel(pred, y, edge_capacity, edge_flow, net_demand, edge_index):
    ei = edge_index.astype(jnp.int32)
    src2 = ei[0].reshape(ROWS, LANE)
    dst2 = ei[1].reshape(ROWS, LANE)
    acc, part = _k1(pred, y, edge_capacity, edge_flow, src2, dst2)
    nd2 = jnp.pad(net_demand, (0, NODES_PAD - N_NODES)).reshape(NODES_PAD // LANE, LANE)
    out = _k2(acc.reshape(2, NODES_PAD // LANE, LANE), nd2,
              part.reshape(2 * NW, 16))
    return out[0, 0]


# speedup vs baseline: 54.2612x; 1.2748x over previous
import functools

import jax
import jax.numpy as jnp
from jax import lax
from jax.experimental import pallas as pl
from jax.experimental.pallas import tpu as pltpu
from jax.experimental.pallas import tpu_sc as plsc

N_NODES = 100000
N_EDGES = 3200000

W_VCR = 1.0
W_FLOW = 0.005
W_CONS = 0.05

TGT_SCALE, TGT_SHIFT = 0.8, 0.5
CAP_SCALE, CAP_SHIFT = 1500.0, 2000.0
FLOW_SCALE, FLOW_SHIFT = 1200.0, 0.0
DEM_SCALE, DEM_SHIFT = 500.0, 0.0

NW = 32
LANE = 128
ROWS = N_EDGES // LANE
CR = 8
CE = CR * LANE
NCHUNKS = ROWS // CR
CHUNK_LO = NCHUNKS // NW
CHUNK_EXTRA = NCHUNKS - CHUNK_LO * NW
NODES_PAD = 100096
SLICE = NODES_PAD // 16


NSETS = 4


def _k1_body(pred_h, y_h, cap_h, flow_h, src_h, dst_h,
             acc_o, part_o, *scr):
    cid = lax.axis_index("c")
    sid = lax.axis_index("s")
    w = cid * 16 + sid
    stage, zbuf, acc_sh = scr[8 * NSETS:8 * NSETS + 3]
    isems = scr[8 * NSETS + 3:8 * NSETS + 3 + NSETS]
    ssems = scr[8 * NSETS + 3 + NSETS:]
    bufs = [tuple(scr[8 * s:8 * s + 8]) + (isems[s], ssems[s])
            for s in range(NSETS)]

    def zstep(i, carry):
        zbuf[pl.ds(i * 16, 16)] = jnp.zeros((16,), jnp.float32)
        return carry

    lax.fori_loop(0, SLICE // 16, zstep, 0)
    pltpu.sync_copy(zbuf, acc_sh.at[pl.ds(sid * SLICE, SLICE)])
    plsc.subcore_barrier()

    wmin = jnp.minimum(w, CHUNK_EXTRA)
    base_chunk = CHUNK_LO * w + wmin
    n_chunks = CHUNK_LO + jnp.where(w < CHUNK_EXTRA, 1, 0)

    def prefetch(g, s):
        pv, yv, cv, fv, _, _, sv, dv, isem, _ = bufs[s]
        e0 = g * CE
        r0 = g * CR
        pltpu.async_copy(pred_h.at[pl.ds(e0, CE)], pv, isem)
        pltpu.async_copy(y_h.at[pl.ds(e0, CE)], yv, isem)
        pltpu.async_copy(cap_h.at[pl.ds(e0, CE)], cv, isem)
        pltpu.async_copy(flow_h.at[pl.ds(e0, CE)], fv, isem)
        pltpu.async_copy(src_h.at[pl.ds(r0, CR)], sv, isem)
        pltpu.async_copy(dst_h.at[pl.ds(r0, CR)], dv, isem)

    def wait_in(s):
        pv, yv, cv, fv, _, _, sv, dv, isem, _ = bufs[s]
        for dst in (pv, yv, cv, fv):
            pltpu.make_async_copy(pred_h.at[pl.ds(0, CE)], dst, isem).wait()
        for dst in (sv, dv):
            pltpu.make_async_copy(src_h.at[pl.ds(0, CR)], dst, isem).wait()

    UNROLL = 4

    def compute(s, mvec, carry):
        pv, yv, cv, fv, pfv, npfv, _, _, _, _ = bufs[s]

        def estep(j, c2):
            for u in range(UNROLL):
                av, af = c2
                o = j * (16 * UNROLL) + u * 16
                p = pv[pl.ds(o, 16)]
                yy = yv[pl.ds(o, 16)]
                cc = cv[pl.ds(o, 16)]
                ff = fv[pl.ds(o, 16)]
                pf = (p * TGT_SCALE + TGT_SHIFT) * (cc * CAP_SCALE + CAP_SHIFT)
                d1 = p - yy
                d2 = pf - (ff * FLOW_SCALE + FLOW_SHIFT)
                if mvec is not None:
                    pf = pf * mvec
                    d1 = d1 * mvec
                    d2 = d2 * mvec
                pfv[pl.ds(o, 16)] = pf
                npfv[pl.ds(o, 16)] = -pf
                c2 = (av + d1 * d1, af + d2 * d2)
            return c2

        return lax.fori_loop(0, CE // (16 * UNROLL), estep, carry)

    def fire(s):
        _, _, _, _, pfv, npfv, sv, dv, _, ssem = bufs[s]
        for k in range(CR):
            pltpu.async_copy(pfv.at[pl.ds(k * LANE, LANE)],
                             acc_sh.at[dv.at[k]], ssem, add=True)
            pltpu.async_copy(npfv.at[pl.ds(k * LANE, LANE)],
                             acc_sh.at[sv.at[k]], ssem, add=True)

    def drain(s):
        _, _, _, _, pfv, npfv, sv, dv, _, ssem = bufs[s]
        for k in range(CR):
            pltpu.make_async_copy(pfv.at[pl.ds(k * LANE, LANE)],
                                  acc_sh.at[dv.at[k]], ssem).wait()
            pltpu.make_async_copy(npfv.at[pl.ds(k * LANE, LANE)],
                                  acc_sh.at[sv.at[k]], ssem).wait()

    zero16 = jnp.zeros((16,), jnp.float32)
    for s in range(NSETS):
        prefetch(base_chunk + s, s)

    wait_in(0)
    carry = compute(0, None, (zero16, zero16))
    fire(0)
    wait_in(1)
    carry = compute(1, None, carry)
    fire(1)

    def body(k, carry):
        for off in (2, 3, 4, 5):
            s = off % NSETS
            c = 4 * k + off
            wait_in(s)
            if off == 5:
                mvec = jnp.zeros((16,), jnp.float32) + jnp.where(
                    c < n_chunks, 1.0, 0.0).astype(jnp.float32)
            else:
                mvec = None
            carry = compute(s, mvec, carry)
            fire(s)
            sd = (s + 2) % NSETS
            drain(sd)
            prefetch(jnp.minimum(base_chunk + c + 2, NCHUNKS - 1), sd)
        return carry

    avcr, aflow = lax.fori_loop(0, 24, body, carry)
    drain(0)
    drain(1)
    wait_in(2)
    wait_in(3)

    stage[...] = avcr
    pltpu.sync_copy(stage, part_o.at[pl.ds(w * 16, 16)])
    stage[...] = aflow
    pltpu.sync_copy(stage, part_o.at[pl.ds((NW + w) * 16, 16)])

    plsc.subcore_barrier()
    pltpu.sync_copy(acc_sh.at[pl.ds(sid * SLICE, SLICE)], zbuf)
    pltpu.sync_copy(zbuf, acc_o.at[pl.ds(cid * NODES_PAD + sid * SLICE, SLICE)])


_k1 = functools.partial(
    pl.kernel,
    mesh=plsc.VectorSubcoreMesh(core_axis_name="c", subcore_axis_name="s",
                                num_cores=2),
    out_type=[
        jax.ShapeDtypeStruct((2 * NODES_PAD,), jnp.float32),
        jax.ShapeDtypeStruct((2 * NW * 16,), jnp.float32),
    ],
    scratch_types=(
        ([pltpu.VMEM((CE,), jnp.float32)] * 6
         + [pltpu.VMEM((CR, LANE), jnp.int32)] * 2) * NSETS
        + [
            pltpu.VMEM((16,), jnp.float32),
            pltpu.VMEM((SLICE,), jnp.float32),
            pltpu.VMEM_SHARED((NODES_PAD,), jnp.float32),
        ]
        + [pltpu.SemaphoreType.DMA] * NSETS
        + [pltpu.SemaphoreType.DMA] * NSETS
    ),
)(_k1_body)


def _k2_body(acc_ref, nd_ref, part_ref, out_ref):
    delta = acc_ref[0] + acc_ref[1]
    rnd = nd_ref[...] * DEM_SCALE + DEM_SHIFT
    cons = jnp.sum(jnp.abs(delta - rnd))
    vcr = jnp.sum(part_ref[0:NW, :])
    flw = jnp.sum(part_ref[NW:2 * NW, :])
    total = (W_VCR * vcr / N_EDGES
             + W_FLOW * flw / N_EDGES
             + W_CONS * cons / N_NODES)
    out_ref[...] = jnp.reshape(total, (1, 1))


_k2 = pl.pallas_call(
    _k2_body,
    out_shape=jax.ShapeDtypeStruct((1, 1), jnp.float32),
)


def kernel(pred, y, edge_capacity, edge_flow, net_demand, edge_index):
    ei = edge_index.astype(jnp.int32)
    src2 = ei[0].reshape(ROWS, LANE)
    dst2 = ei[1].reshape(ROWS, LANE)
    acc, part = _k1(pred, y, edge_capacity, edge_flow, src2, dst2)
    nd2 = jnp.pad(net_demand, (0, NODES_PAD - N_NODES)).reshape(NODES_PAD // LANE, LANE)
    out = _k2(acc.reshape(2, NODES_PAD // LANE, LANE), nd2,
              part.reshape(2 * NW, 16))
    return out[0, 0]


# speedup vs baseline: 54.3599x; 1.0018x over previous
import functools

import jax
import jax.numpy as jnp
from jax import lax
from jax.experimental import pallas as pl
from jax.experimental.pallas import tpu as pltpu
from jax.experimental.pallas import tpu_sc as plsc

N_NODES = 100000
N_EDGES = 3200000

W_VCR = 1.0
W_FLOW = 0.005
W_CONS = 0.05

TGT_SCALE, TGT_SHIFT = 0.8, 0.5
CAP_SCALE, CAP_SHIFT = 1500.0, 2000.0
FLOW_SCALE, FLOW_SHIFT = 1200.0, 0.0
DEM_SCALE, DEM_SHIFT = 500.0, 0.0

NW = 32
LANE = 128
ROWS = N_EDGES // LANE
CR = 8
CE = CR * LANE
NCHUNKS = ROWS // CR
CHUNK_LO = NCHUNKS // NW
CHUNK_EXTRA = NCHUNKS - CHUNK_LO * NW
NODES_PAD = 100096
SLICE = NODES_PAD // 16


NSETS = 4


def _k1_body(pred_h, y_h, cap_h, flow_h, src_h, dst_h,
             acc_o, part_o, *scr):
    cid = lax.axis_index("c")
    sid = lax.axis_index("s")
    w = cid * 16 + sid
    stage, zbuf, acc_sh = scr[8 * NSETS:8 * NSETS + 3]
    isems = scr[8 * NSETS + 3:8 * NSETS + 3 + NSETS]
    ssems = scr[8 * NSETS + 3 + NSETS:]
    bufs = [tuple(scr[8 * s:8 * s + 8]) + (isems[s], ssems[s])
            for s in range(NSETS)]

    def zstep(i, carry):
        zbuf[pl.ds(i * 16, 16)] = jnp.zeros((16,), jnp.float32)
        return carry

    lax.fori_loop(0, SLICE // 16, zstep, 0)
    pltpu.sync_copy(zbuf, acc_sh.at[pl.ds(sid * SLICE, SLICE)])
    plsc.subcore_barrier()

    wmin = jnp.minimum(w, CHUNK_EXTRA)
    base_chunk = CHUNK_LO * w + wmin
    n_chunks = CHUNK_LO + jnp.where(w < CHUNK_EXTRA, 1, 0)

    def prefetch(g, s):
        pv, yv, cv, fv, _, _, sv, dv, isem, _ = bufs[s]
        e0 = g * CE
        r0 = g * CR
        pltpu.async_copy(pred_h.at[pl.ds(e0, CE)], pv, isem)
        pltpu.async_copy(y_h.at[pl.ds(e0, CE)], yv, isem)
        pltpu.async_copy(cap_h.at[pl.ds(e0, CE)], cv, isem)
        pltpu.async_copy(flow_h.at[pl.ds(e0, CE)], fv, isem)
        pltpu.async_copy(src_h.at[pl.ds(e0, CE)], sv, isem)
        pltpu.async_copy(dst_h.at[pl.ds(e0, CE)], dv, isem)

    def wait_in(s):
        pv, yv, cv, fv, _, _, sv, dv, isem, _ = bufs[s]
        for dst in (pv, yv, cv, fv):
            pltpu.make_async_copy(pred_h.at[pl.ds(0, CE)], dst, isem).wait()
        for dst in (sv, dv):
            pltpu.make_async_copy(src_h.at[pl.ds(0, CE)], dst, isem).wait()

    def compute(s, mvec, carry):
        pv, yv, cv, fv, pfv, npfv, _, _, _, _ = bufs[s]

        def estep(j, c2):
            for u in range(LANE // 16):
                av, af = c2
                o = j * LANE + u * 16
                p = pv[pl.ds(o, 16)]
                yy = yv[pl.ds(o, 16)]
                cc = cv[pl.ds(o, 16)]
                ff = fv[pl.ds(o, 16)]
                pf = (p * TGT_SCALE + TGT_SHIFT) * (cc * CAP_SCALE + CAP_SHIFT)
                d1 = p - yy
                d2 = pf - (ff * FLOW_SCALE + FLOW_SHIFT)
                if mvec is not None:
                    pf = pf * mvec
                    d1 = d1 * mvec
                    d2 = d2 * mvec
                pfv[pl.ds(o, 16)] = pf
                npfv[pl.ds(o, 16)] = -pf
                c2 = (av + d1 * d1, af + d2 * d2)
            return c2

        return lax.fori_loop(0, CR, estep, carry)

    def fire(s):
        _, _, _, _, pfv, npfv, sv, dv, _, ssem = bufs[s]
        pltpu.async_copy(pfv, acc_sh.at[dv], ssem, add=True)
        pltpu.async_copy(npfv, acc_sh.at[sv], ssem, add=True)

    def drain(s):
        _, _, _, _, pfv, npfv, sv, dv, _, ssem = bufs[s]
        pltpu.make_async_copy(pfv, acc_sh.at[dv], ssem).wait()
        pltpu.make_async_copy(npfv, acc_sh.at[sv], ssem).wait()

    zero16 = jnp.zeros((16,), jnp.float32)
    for s in range(NSETS):
        prefetch(base_chunk + s, s)

    wait_in(0)
    carry = compute(0, None, (zero16, zero16))
    fire(0)
    wait_in(1)
    carry = compute(1, None, carry)
    fire(1)

    def body(k, carry):
        for off in (2, 3, 4, 5):
            s = off % NSETS
            c = 4 * k + off
            wait_in(s)
            if off == 5:
                mvec = jnp.zeros((16,), jnp.float32) + jnp.where(
                    c < n_chunks, 1.0, 0.0).astype(jnp.float32)
            else:
                mvec = None
            carry = compute(s, mvec, carry)
            fire(s)
            sd = (s + 2) % NSETS
            drain(sd)
            prefetch(jnp.minimum(base_chunk + c + 2, NCHUNKS - 1), sd)
        return carry

    avcr, aflow = lax.fori_loop(0, 24, body, carry)
    drain(0)
    drain(1)
    wait_in(2)
    wait_in(3)

    stage[...] = avcr
    pltpu.sync_copy(stage, part_o.at[pl.ds(w * 16, 16)])
    stage[...] = aflow
    pltpu.sync_copy(stage, part_o.at[pl.ds((NW + w) * 16, 16)])

    plsc.subcore_barrier()
    pltpu.sync_copy(acc_sh.at[pl.ds(sid * SLICE, SLICE)], zbuf)
    pltpu.sync_copy(zbuf, acc_o.at[pl.ds(cid * NODES_PAD + sid * SLICE, SLICE)])


_k1 = functools.partial(
    pl.kernel,
    mesh=plsc.VectorSubcoreMesh(core_axis_name="c", subcore_axis_name="s",
                                num_cores=2),
    out_type=[
        jax.ShapeDtypeStruct((2 * NODES_PAD,), jnp.float32),
        jax.ShapeDtypeStruct((2 * NW * 16,), jnp.float32),
    ],
    scratch_types=(
        ([pltpu.VMEM((CE,), jnp.float32)] * 6
         + [pltpu.VMEM((CE,), jnp.int32)] * 2) * NSETS
        + [
            pltpu.VMEM((16,), jnp.float32),
            pltpu.VMEM((SLICE,), jnp.float32),
            pltpu.VMEM_SHARED((NODES_PAD,), jnp.float32),
        ]
        + [pltpu.SemaphoreType.DMA] * NSETS
        + [pltpu.SemaphoreType.DMA] * NSETS
    ),
)(_k1_body)


def _k2_body(acc_ref, nd_ref, part_ref, out_ref):
    delta = acc_ref[0] + acc_ref[1]
    rnd = nd_ref[...] * DEM_SCALE + DEM_SHIFT
    cons = jnp.sum(jnp.abs(delta - rnd))
    vcr = jnp.sum(part_ref[0:NW, :])
    flw = jnp.sum(part_ref[NW:2 * NW, :])
    total = (W_VCR * vcr / N_EDGES
             + W_FLOW * flw / N_EDGES
             + W_CONS * cons / N_NODES)
    out_ref[...] = jnp.reshape(total, (1, 1))


_k2 = pl.pallas_call(
    _k2_body,
    out_shape=jax.ShapeDtypeStruct((1, 1), jnp.float32),
)


def kernel(pred, y, edge_capacity, edge_flow, net_demand, edge_index):
    ei = edge_index.astype(jnp.int32)
    acc, part = _k1(pred, y, edge_capacity, edge_flow, ei[0], ei[1])
    nd2 = jnp.pad(net_demand, (0, NODES_PAD - N_NODES)).reshape(NODES_PAD // LANE, LANE)
    out = _k2(acc.reshape(2, NODES_PAD // LANE, LANE), nd2,
              part.reshape(2 * NW, 16))
    return out[0, 0]


# speedup vs baseline: 54.5678x; 1.0038x over previous
import functools

import jax
import jax.numpy as jnp
from jax import lax
from jax.experimental import pallas as pl
from jax.experimental.pallas import tpu as pltpu
from jax.experimental.pallas import tpu_sc as plsc

N_NODES = 100000
N_EDGES = 3200000

W_VCR = 1.0
W_FLOW = 0.005
W_CONS = 0.05

TGT_SCALE, TGT_SHIFT = 0.8, 0.5
CAP_SCALE, CAP_SHIFT = 1500.0, 2000.0
FLOW_SCALE, FLOW_SHIFT = 1200.0, 0.0
DEM_SCALE, DEM_SHIFT = 500.0, 0.0

NW = 32
LANE = 128
CR = 40
CE = CR * LANE
NCHUNKS = N_EDGES // CE
CHUNK_LO = NCHUNKS // NW
CHUNK_EXTRA = NCHUNKS - CHUNK_LO * NW
NBLK = CHUNK_LO + 1
NODES_PAD = 100096
SLICE = NODES_PAD // 16
NSETS = 3


def _k1_body(pred_h, y_h, cap_h, flow_h, eix_h, acc_o, part_o, *scr):
    cid = lax.axis_index("c")
    sid = lax.axis_index("s")
    w = cid * 16 + sid
    stage, acc_sh = scr[8 * NSETS], scr[8 * NSETS + 1]
    isems = scr[8 * NSETS + 2:8 * NSETS + 2 + NSETS]
    ssems = scr[8 * NSETS + 2 + NSETS:]
    bufs = [tuple(scr[8 * s:8 * s + 8]) + (isems[s], ssems[s])
            for s in range(NSETS)]
    pfv0 = bufs[0][4]

    def zstep(j, carry):
        for u in range(LANE // 16):
            pfv0[pl.ds(j * LANE + u * 16, 16)] = jnp.zeros((16,), jnp.float32)
        return carry

    lax.fori_loop(0, CR, zstep, 0)
    pltpu.sync_copy(pfv0, acc_sh.at[pl.ds(sid * SLICE, CE)])
    pltpu.sync_copy(pfv0.at[pl.ds(0, SLICE - CE)],
                    acc_sh.at[pl.ds(sid * SLICE + CE, SLICE - CE)])
    plsc.subcore_barrier()

    wmin = jnp.minimum(w, CHUNK_EXTRA)
    base_chunk = CHUNK_LO * w + wmin
    n_chunks = CHUNK_LO + jnp.where(w < CHUNK_EXTRA, 1, 0)

    def prefetch(g, s):
        pv, yv, cv, fv, _, _, sv, dv, isem, _ = bufs[s]
        e0 = g * CE
        pltpu.async_copy(pred_h.at[pl.ds(e0, CE)], pv, isem)
        pltpu.async_copy(y_h.at[pl.ds(e0, CE)], yv, isem)
        pltpu.async_copy(cap_h.at[pl.ds(e0, CE)], cv, isem)
        pltpu.async_copy(flow_h.at[pl.ds(e0, CE)], fv, isem)
        pltpu.async_copy(eix_h.at[pl.ds(e0, CE)], sv, isem)
        pltpu.async_copy(eix_h.at[pl.ds(N_EDGES + e0, CE)], dv, isem)

    def wait_in(s):
        pv, yv, cv, fv, _, _, sv, dv, isem, _ = bufs[s]
        for dst in (pv, yv, cv, fv, sv, dv):
            pltpu.make_async_copy(pred_h.at[pl.ds(0, CE)], dst, isem).wait()

    def compute(s, mvec, carry):
        pv, yv, cv, fv, pfv, npfv, _, _, _, _ = bufs[s]

        def estep(j, c2):
            for u in range(LANE // 16):
                av, af = c2
                o = j * LANE + u * 16
                p = pv[pl.ds(o, 16)]
                yy = yv[pl.ds(o, 16)]
                cc = cv[pl.ds(o, 16)]
                ff = fv[pl.ds(o, 16)]
                pf = (p * TGT_SCALE + TGT_SHIFT) * (cc * CAP_SCALE + CAP_SHIFT)
                d1 = p - yy
                d2 = pf - (ff * FLOW_SCALE + FLOW_SHIFT)
                if mvec is not None:
                    pf = pf * mvec
                    d1 = d1 * mvec
                    d2 = d2 * mvec
                pfv[pl.ds(o, 16)] = pf
                npfv[pl.ds(o, 16)] = -pf
                c2 = (av + d1 * d1, af + d2 * d2)
            return c2

        return lax.fori_loop(0, CR, estep, carry)

    def fire(s):
        _, _, _, _, pfv, npfv, sv, dv, _, ssem = bufs[s]
        pltpu.async_copy(pfv, acc_sh.at[dv], ssem, add=True)
        pltpu.async_copy(npfv, acc_sh.at[sv], ssem, add=True)

    def drain(s):
        _, _, _, _, pfv, npfv, sv, dv, _, ssem = bufs[s]
        pltpu.make_async_copy(pfv, acc_sh.at[dv], ssem).wait()
        pltpu.make_async_copy(npfv, acc_sh.at[sv], ssem).wait()

    zero16 = jnp.zeros((16,), jnp.float32)
    for s in range(NSETS):
        prefetch(base_chunk + s, s)

    wait_in(0)
    carry = compute(0, None, (zero16, zero16))
    fire(0)
    wait_in(1)
    carry = compute(1, None, carry)
    fire(1)

    def body(k, carry):
        for off in (2, 3, 4):
            s = off % NSETS
            c = 3 * k + off
            wait_in(s)
            if off == 4:
                mvec = jnp.zeros((16,), jnp.float32) + jnp.where(
                    c < n_chunks, 1.0, 0.0).astype(jnp.float32)
            else:
                mvec = None
            carry = compute(s, mvec, carry)
            fire(s)
            sd = (s + 1) % NSETS
            drain(sd)
            prefetch(jnp.minimum(base_chunk + c + 1, NCHUNKS - 1), sd)
        return carry

    avcr, aflow = lax.fori_loop(0, (NBLK - 2) // 3, body, carry)
    drain(0)
    drain(1)
    wait_in(2)

    stage[...] = avcr
    pltpu.sync_copy(stage, part_o.at[pl.ds(w * 16, 16)])
    stage[...] = aflow
    pltpu.sync_copy(stage, part_o.at[pl.ds((NW + w) * 16, 16)])

    plsc.subcore_barrier()
    obase = cid * NODES_PAD + sid * SLICE
    pltpu.sync_copy(acc_sh.at[pl.ds(sid * SLICE, CE)], pfv0)
    pltpu.sync_copy(pfv0, acc_o.at[pl.ds(obase, CE)])
    pltpu.sync_copy(acc_sh.at[pl.ds(sid * SLICE + CE, SLICE - CE)],
                    pfv0.at[pl.ds(0, SLICE - CE)])
    pltpu.sync_copy(pfv0.at[pl.ds(0, SLICE - CE)],
                    acc_o.at[pl.ds(obase + CE, SLICE - CE)])


_k1 = functools.partial(
    pl.kernel,
    mesh=plsc.VectorSubcoreMesh(core_axis_name="c", subcore_axis_name="s",
                                num_cores=2),
    out_type=[
        jax.ShapeDtypeStruct((2 * NODES_PAD,), jnp.float32),
        jax.ShapeDtypeStruct((2 * NW * 16,), jnp.float32),
    ],
    scratch_types=(
        ([pltpu.VMEM((CE,), jnp.float32)] * 6
         + [pltpu.VMEM((CE,), jnp.int32)] * 2) * NSETS
        + [
            pltpu.VMEM((16,), jnp.float32),
            pltpu.VMEM_SHARED((NODES_PAD,), jnp.float32),
        ]
        + [pltpu.SemaphoreType.DMA] * NSETS
        + [pltpu.SemaphoreType.DMA] * NSETS
    ),
)(_k1_body)


def _k2_body(acc_ref, nd_ref, part_ref, out_ref):
    delta = acc_ref[0] + acc_ref[1]
    rnd = nd_ref[...] * DEM_SCALE + DEM_SHIFT
    cons = jnp.sum(jnp.abs(delta - rnd))
    vcr = jnp.sum(part_ref[0:NW, :])
    flw = jnp.sum(part_ref[NW:2 * NW, :])
    total = (W_VCR * vcr / N_EDGES
             + W_FLOW * flw / N_EDGES
             + W_CONS * cons / N_NODES)
    out_ref[...] = jnp.reshape(total, (1, 1))


_k2 = pl.pallas_call(
    _k2_body,
    out_shape=jax.ShapeDtypeStruct((1, 1), jnp.float32),
)


def kernel(pred, y, edge_capacity, edge_flow, net_demand, edge_index):
    eix = edge_index.reshape(2 * N_EDGES)
    acc, part = _k1(pred, y, edge_capacity, edge_flow, eix)
    nd2 = jnp.pad(net_demand, (0, NODES_PAD - N_NODES)).reshape(NODES_PAD // LANE, LANE)
    out = _k2(acc.reshape(2, NODES_PAD // LANE, LANE), nd2,
              part.reshape(2 * NW, 16))
    return out[0, 0]


# speedup vs baseline: 54.6915x; 1.0023x over previous
import functools

import jax
import jax.numpy as jnp
from jax import lax
from jax.experimental import pallas as pl
from jax.experimental.pallas import tpu as pltpu
from jax.experimental.pallas import tpu_sc as plsc

N_NODES = 100000
N_EDGES = 3200000

W_VCR = 1.0
W_FLOW = 0.005
W_CONS = 0.05

TGT_SCALE, TGT_SHIFT = 0.8, 0.5
CAP_SCALE, CAP_SHIFT = 1500.0, 2000.0
FLOW_SCALE, FLOW_SHIFT = 1200.0, 0.0
DEM_SCALE, DEM_SHIFT = 500.0, 0.0

NW = 32
LANE = 128
CR = 40
CE = CR * LANE
NCHUNKS = N_EDGES // CE
CHUNK_LO = NCHUNKS // NW
CHUNK_EXTRA = NCHUNKS - CHUNK_LO * NW
NBLK = CHUNK_LO + 1
NODES_PAD = 100096
SLICE = NODES_PAD // 16
NSETS = 3


def _k1_body(pred_h, y_h, cap_h, flow_h, eix_h, acc_o, part_o, *scr):
    cid = lax.axis_index("c")
    sid = lax.axis_index("s")
    w = cid * 16 + sid
    stage, acc_sh = scr[8 * NSETS], scr[8 * NSETS + 1]
    isems = scr[8 * NSETS + 2:8 * NSETS + 2 + NSETS]
    ssems = scr[8 * NSETS + 2 + NSETS:]
    bufs = [tuple(scr[8 * s:8 * s + 8]) + (isems[s], ssems[s])
            for s in range(NSETS)]
    pfv0 = bufs[0][4]

    def zstep(j, carry):
        for u in range(LANE // 16):
            pfv0[pl.ds(j * LANE + u * 16, 16)] = jnp.zeros((16,), jnp.float32)
        return carry

    lax.fori_loop(0, CR, zstep, 0)
    pltpu.sync_copy(pfv0, acc_sh.at[pl.ds(sid * SLICE, CE)])
    pltpu.sync_copy(pfv0.at[pl.ds(0, SLICE - CE)],
                    acc_sh.at[pl.ds(sid * SLICE + CE, SLICE - CE)])
    plsc.subcore_barrier()

    wmin = jnp.minimum(w, CHUNK_EXTRA)
    base_chunk = CHUNK_LO * w + wmin
    n_chunks = CHUNK_LO + jnp.where(w < CHUNK_EXTRA, 1, 0)

    def prefetch(g, s):
        pv, yv, cv, fv, _, _, sv, dv, isem, _ = bufs[s]
        e0 = g * CE
        pltpu.async_copy(pred_h.at[pl.ds(e0, CE)], pv, isem)
        pltpu.async_copy(y_h.at[pl.ds(e0, CE)], yv, isem)
        pltpu.async_copy(cap_h.at[pl.ds(e0, CE)], cv, isem)
        pltpu.async_copy(flow_h.at[pl.ds(e0, CE)], fv, isem)
        pltpu.async_copy(eix_h.at[pl.ds(e0, CE)], sv, isem)
        pltpu.async_copy(eix_h.at[pl.ds(N_EDGES + e0, CE)], dv, isem)

    def wait_in(s):
        pv, yv, cv, fv, _, _, sv, dv, isem, _ = bufs[s]
        for dst in (pv, yv, cv, fv, sv, dv):
            pltpu.make_async_copy(pred_h.at[pl.ds(0, CE)], dst, isem).wait()

    def compute(s, mvec, carry):
        pv, yv, cv, fv, pfv, npfv, _, _, _, _ = bufs[s]

        def estep(j, c2):
            for u in range(LANE // 16):
                av, af = c2
                o = j * LANE + u * 16
                p = pv[pl.ds(o, 16)]
                yy = yv[pl.ds(o, 16)]
                cc = cv[pl.ds(o, 16)]
                ff = fv[pl.ds(o, 16)]
                pf = (p * TGT_SCALE + TGT_SHIFT) * (cc * CAP_SCALE + CAP_SHIFT)
                d1 = p - yy
                d2 = pf - (ff * FLOW_SCALE + FLOW_SHIFT)
                if mvec is not None:
                    pf = pf * mvec
                    d1 = d1 * mvec
                    d2 = d2 * mvec
                pfv[pl.ds(o, 16)] = pf
                npfv[pl.ds(o, 16)] = -pf
                c2 = (av + d1 * d1, af + d2 * d2)
            return c2

        return lax.fori_loop(0, CR, estep, carry)

    def fire(s):
        _, _, _, _, pfv, npfv, sv, dv, _, ssem = bufs[s]
        pltpu.async_copy(pfv, acc_sh.at[dv], ssem, add=True)
        pltpu.async_copy(npfv, acc_sh.at[sv], ssem, add=True)

    def drain(s):
        _, _, _, _, pfv, npfv, sv, dv, _, ssem = bufs[s]
        pltpu.make_async_copy(pfv, acc_sh.at[dv], ssem).wait()
        pltpu.make_async_copy(npfv, acc_sh.at[sv], ssem).wait()

    zero16 = jnp.zeros((16,), jnp.float32)
    for s in range(NSETS):
        prefetch(base_chunk + s, s)

    wait_in(0)
    carry = compute(0, None, (zero16, zero16))
    fire(0)
    wait_in(1)
    carry = compute(1, None, carry)
    fire(1)

    def body(k, carry):
        for off in (2, 3, 4):
            s = off % NSETS
            c = 3 * k + off
            wait_in(s)
            sd = (s + 1) % NSETS
            drain(sd)
            prefetch(jnp.minimum(base_chunk + c + 1, NCHUNKS - 1), sd)
            if off == 4:
                mvec = jnp.zeros((16,), jnp.float32) + jnp.where(
                    c < n_chunks, 1.0, 0.0).astype(jnp.float32)
            else:
                mvec = None
            carry = compute(s, mvec, carry)
            fire(s)
        return carry

    avcr, aflow = lax.fori_loop(0, (NBLK - 2) // 3, body, carry)
    drain(0)
    drain(1)
    wait_in(2)

    stage[...] = avcr
    pltpu.sync_copy(stage, part_o.at[pl.ds(w * 16, 16)])
    stage[...] = aflow
    pltpu.sync_copy(stage, part_o.at[pl.ds((NW + w) * 16, 16)])

    plsc.subcore_barrier()
    obase = cid * NODES_PAD + sid * SLICE
    pltpu.sync_copy(acc_sh.at[pl.ds(sid * SLICE, CE)], pfv0)
    pltpu.sync_copy(pfv0, acc_o.at[pl.ds(obase, CE)])
    pltpu.sync_copy(acc_sh.at[pl.ds(sid * SLICE + CE, SLICE - CE)],
                    pfv0.at[pl.ds(0, SLICE - CE)])
    pltpu.sync_copy(pfv0.at[pl.ds(0, SLICE - CE)],
                    acc_o.at[pl.ds(obase + CE, SLICE - CE)])


_k1 = functools.partial(
    pl.kernel,
    mesh=plsc.VectorSubcoreMesh(core_axis_name="c", subcore_axis_name="s",
                                num_cores=2),
    out_type=[
        jax.ShapeDtypeStruct((2 * NODES_PAD,), jnp.float32),
        jax.ShapeDtypeStruct((2 * NW * 16,), jnp.float32),
    ],
    scratch_types=(
        ([pltpu.VMEM((CE,), jnp.float32)] * 6
         + [pltpu.VMEM((CE,), jnp.int32)] * 2) * NSETS
        + [
            pltpu.VMEM((16,), jnp.float32),
            pltpu.VMEM_SHARED((NODES_PAD,), jnp.float32),
        ]
        + [pltpu.SemaphoreType.DMA] * NSETS
        + [pltpu.SemaphoreType.DMA] * NSETS
    ),
)(_k1_body)


def _k2_body(acc_ref, nd_ref, part_ref, out_ref):
    delta = acc_ref[0] + acc_ref[1]
    rnd = nd_ref[...] * DEM_SCALE + DEM_SHIFT
    cons = jnp.sum(jnp.abs(delta - rnd))
    vcr = jnp.sum(part_ref[0:NW, :])
    flw = jnp.sum(part_ref[NW:2 * NW, :])
    total = (W_VCR * vcr / N_EDGES
             + W_FLOW * flw / N_EDGES
             + W_CONS * cons / N_NODES)
    out_ref[...] = jnp.reshape(total, (1, 1))


_k2 = pl.pallas_call(
    _k2_body,
    out_shape=jax.ShapeDtypeStruct((1, 1), jnp.float32),
)


def kernel(pred, y, edge_capacity, edge_flow, net_demand, edge_index):
    eix = edge_index.reshape(2 * N_EDGES)
    acc, part = _k1(pred, y, edge_capacity, edge_flow, eix)
    nd2 = jnp.pad(net_demand, (0, NODES_PAD - N_NODES)).reshape(NODES_PAD // LANE, LANE)
    out = _k2(acc.reshape(2, NODES_PAD // LANE, LANE), nd2,
              part.reshape(2 * NW, 16))
    return out[0, 0]
